# baseline (speedup 1.0000x reference)
import jax
import jax.numpy as jnp
from jax.experimental import pallas as pl
from jax.experimental.pallas import tpu as pltpu

_B, _D1, _F = 4096, 256, 128
_BLK = 512


def _pairwise_max_kernel(x0_ref, x1_ref, x2_ref, out_ref):
    x0 = x0_ref[...]
    x1 = x1_ref[...]
    mx = jnp.max(x1, axis=1, keepdims=True)
    mn = jnp.min(x1, axis=1, keepdims=True)
    out_ref[:, :_D1] = jnp.maximum(x0 * mx, x0 * mn)
    out_ref[:, _D1:] = x2_ref[...]


def kernel(x0, x1, x2):
    B, D1 = x0.shape
    F = x2.shape[1]
    grid = (B // _BLK,)
    return pl.pallas_call(
        _pairwise_max_kernel,
        grid=grid,
        in_specs=[
            pl.BlockSpec((_BLK, D1), lambda i: (i, 0)),
            pl.BlockSpec((_BLK, x1.shape[1]), lambda i: (i, 0)),
            pl.BlockSpec((_BLK, F), lambda i: (i, 0)),
        ],
        out_specs=pl.BlockSpec((_BLK, D1 + F), lambda i: (i, 0)),
        out_shape=jax.ShapeDtypeStruct((B, D1 + F), x0.dtype),
        compiler_params=pltpu.CompilerParams(
            dimension_semantics=("parallel",),
        ),
    )(x0, x1, x2)


# speedup vs baseline: 1.2771x; 1.2771x over previous
import jax
import jax.numpy as jnp
from jax.experimental import pallas as pl
from jax.experimental.pallas import tpu as pltpu

_B, _D1, _F = 4096, 256, 128
_BLK = 1024


def _pairwise_max_kernel(x0_ref, x1_ref, x2_ref, out_ref):
    x0 = x0_ref[...]
    x1 = x1_ref[...]
    mx = jnp.max(x1, axis=1, keepdims=True)
    mn = jnp.min(x1, axis=1, keepdims=True)
    out_ref[:, :_D1] = jnp.maximum(x0 * mx, x0 * mn)
    out_ref[:, _D1:] = x2_ref[...]


def kernel(x0, x1, x2):
    B, D1 = x0.shape
    F = x2.shape[1]
    grid = (B // _BLK,)
    return pl.pallas_call(
        _pairwise_max_kernel,
        grid=grid,
        in_specs=[
            pl.BlockSpec((_BLK, D1), lambda i: (i, 0)),
            pl.BlockSpec((_BLK, x1.shape[1]), lambda i: (i, 0)),
            pl.BlockSpec((_BLK, F), lambda i: (i, 0)),
        ],
        out_specs=pl.BlockSpec((_BLK, D1 + F), lambda i: (i, 0)),
        out_shape=jax.ShapeDtypeStruct((B, D1 + F), x0.dtype),
        compiler_params=pltpu.CompilerParams(
            dimension_semantics=("parallel",),
        ),
    )(x0, x1, x2)


# speedup vs baseline: 1.5239x; 1.1932x over previous
import jax
import jax.numpy as jnp
from jax.experimental import pallas as pl
from jax.experimental.pallas import tpu as pltpu

_B, _D1, _F = 4096, 256, 128
_BLK = 2048


def _pairwise_max_kernel(x0_ref, x1_ref, x2_ref, out_ref):
    x0 = x0_ref[...]
    x1 = x1_ref[...]
    mx = jnp.max(x1, axis=1, keepdims=True)
    mn = jnp.min(x1, axis=1, keepdims=True)
    out_ref[:, :_D1] = jnp.maximum(x0 * mx, x0 * mn)
    out_ref[:, _D1:] = x2_ref[...]


def kernel(x0, x1, x2):
    B, D1 = x0.shape
    F = x2.shape[1]
    grid = (B // _BLK,)
    return pl.pallas_call(
        _pairwise_max_kernel,
        grid=grid,
        in_specs=[
            pl.BlockSpec((_BLK, D1), lambda i: (i, 0)),
            pl.BlockSpec((_BLK, x1.shape[1]), lambda i: (i, 0)),
            pl.BlockSpec((_BLK, F), lambda i: (i, 0)),
        ],
        out_specs=pl.BlockSpec((_BLK, D1 + F), lambda i: (i, 0)),
        out_shape=jax.ShapeDtypeStruct((B, D1 + F), x0.dtype),
        compiler_params=pltpu.CompilerParams(
            dimension_semantics=("parallel",),
        ),
    )(x0, x1, x2)
